# baseline (speedup 1.0000x reference)
import functools

import jax
import jax.numpy as jnp
from jax import lax
from jax.experimental import pallas as pl
from jax.experimental.pallas import tpu as pltpu
from jax.experimental.pallas import tpu_sc as plsc

N = 10000
E = 320000
D = 128

NC = 2
NS = 16
NW = NC * NS
CHUNK = 128
CPW = 80
EPW = CPW * CHUNK
EPAD = NW * EPW
NPAD = 10112
RPS = NPAD // NS

_f32 = jnp.float32


def _deg_body(col2d, zeros_n, out, idx_v, ones_v, deg_s):
    cid = lax.axis_index("c")
    sid = lax.axis_index("s")
    wid = sid * NC + cid
    pltpu.sync_copy(col2d.at[pl.ds(wid * CPW, CPW)], idx_v)
    for k in range(CHUNK // 16):
        ones_v[pl.ds(k * 16, 16)] = jnp.full((16,), 1.0, _f32)

    @pl.when(sid == 0)
    def _():
        pltpu.sync_copy(zeros_n, deg_s)

    plsc.subcore_barrier()

    def body(j, c):
        pltpu.sync_copy(ones_v, deg_s.at[idx_v.at[j]], add=True)
        return c

    lax.fori_loop(0, CPW, body, 0)
    plsc.subcore_barrier()

    @pl.when(sid == 0)
    def _():
        pltpu.sync_copy(deg_s, out.at[cid])


_k1 = functools.partial(
    pl.kernel,
    out_type=jax.ShapeDtypeStruct((NC, NPAD), _f32),
    mesh=plsc.VectorSubcoreMesh(core_axis_name="c", subcore_axis_name="s"),
    scratch_types=[
        pltpu.VMEM((CPW, CHUNK), jnp.int32),
        pltpu.VMEM((CHUNK,), _f32),
        pltpu.VMEM_SHARED((NPAD,), _f32),
    ],
)(_deg_body)


def _tc1_body(x_ref, w1_ref, dp_ref, dis_ref, xws_ref):
    deg = dp_ref[0:1, :] + dp_ref[1:2, :] + 1.0
    dis_row = lax.rsqrt(deg)
    dis_ref[...] = dis_row.reshape(NPAD)
    dis_col = jnp.transpose(dis_row)
    xs = x_ref[...] * dis_col[:N]
    xw = jnp.dot(xs, w1_ref[...], preferred_element_type=_f32)
    xws_ref[0:N, :] = xw
    xws_ref[N:NPAD, :] = jnp.zeros((NPAD - N, D), _f32)


_k2 = pl.pallas_call(
    _tc1_body,
    out_shape=(
        jax.ShapeDtypeStruct((NPAD,), _f32),
        jax.ShapeDtypeStruct((NPAD, D), _f32),
    ),
)


_CPH = CPW // 2


def _edge_body(row2d, col2d, xws, dis_h, zeros_nd, zeros_n,
               acc_out, g_out,
               ridx, cidx, rows0, rows1, gvals, acc_s, g_s, dis_spm,
               sem0, sem1, ssem0, ssem1, gsem, gsem2):
    cid = lax.axis_index("c")
    sid = lax.axis_index("s")
    wid = sid * NC + cid
    base = wid * CPW
    pltpu.sync_copy(zeros_nd.at[pl.ds(sid * RPS, RPS)],
                    acc_s.at[pl.ds(sid * RPS, RPS)])

    @pl.when(sid == 0)
    def _():
        pltpu.sync_copy(zeros_n, g_s)
        pltpu.sync_copy(dis_h, dis_spm)

    plsc.subcore_barrier()

    rows_b = (rows0, rows1)
    sem_b = (sem0, sem1)
    ssem_b = (ssem0, ssem1)

    def _gather(j, b):
        pltpu.async_copy(xws.at[ridx.at[j]], rows_b[b], sem_b[b])

    def _wait_gather(j, b):
        pltpu.make_async_copy(xws.at[ridx.at[j]], rows_b[b], sem_b[b]).wait()

    def _scatter(j, b):
        pltpu.async_copy(rows_b[b], acc_s.at[cidx.at[j]], ssem_b[b],
                         add=True)

    def _wait_scatter(j, b):
        pltpu.make_async_copy(rows_b[b], acc_s.at[cidx.at[j]],
                              ssem_b[b]).wait()

    for h in range(2):
        pltpu.sync_copy(row2d.at[pl.ds(base + h * _CPH, _CPH)], ridx)
        pltpu.sync_copy(col2d.at[pl.ds(base + h * _CPH, _CPH)], cidx)
        _gather(0, 0)
        _gather(1, 1)

        def _g_issue(j, gs):
            pltpu.async_copy(dis_spm.at[cidx.at[j]], gvals.at[j], gs)

        def _g_drain(j, gs):
            pltpu.make_async_copy(dis_spm.at[cidx.at[j]], gvals.at[j],
                                  gs).wait()
            pltpu.sync_copy(gvals.at[j], g_s.at[ridx.at[j]], add=True)

        def body(t, c):
            j0 = 2 * t
            _wait_gather(j0, 0)
            _scatter(j0, 0)
            _wait_gather(j0 + 1, 1)
            _scatter(j0 + 1, 1)

            @pl.when(t > 0)
            def _():
                _g_drain(j0 - 2, gsem)
                _g_drain(j0 - 1, gsem2)

            _g_issue(j0, gsem)
            _g_issue(j0 + 1, gsem2)

            @pl.when(j0 + 2 < _CPH)
            def _():
                _wait_scatter(j0, 0)
                _gather(j0 + 2, 0)
                _wait_scatter(j0 + 1, 1)
                _gather(j0 + 3, 1)

            return c

        lax.fori_loop(0, _CPH // 2, body, 0)
        _wait_scatter(_CPH - 2, 0)
        _wait_scatter(_CPH - 1, 1)
        _g_drain(_CPH - 2, gsem)
        _g_drain(_CPH - 1, gsem2)

    plsc.subcore_barrier()

    pltpu.sync_copy(acc_s.at[pl.ds(sid * RPS, RPS)],
                    acc_out.at[cid, pl.ds(sid * RPS, RPS)])

    @pl.when(sid == 0)
    def _():
        pltpu.sync_copy(g_s, g_out.at[cid])


_k3 = functools.partial(
    pl.kernel,
    out_type=(
        jax.ShapeDtypeStruct((NC, NPAD, D), _f32),
        jax.ShapeDtypeStruct((NC, NPAD), _f32),
    ),
    mesh=plsc.VectorSubcoreMesh(core_axis_name="c", subcore_axis_name="s"),
    scratch_types=[
        pltpu.VMEM((_CPH, CHUNK), jnp.int32),
        pltpu.VMEM((_CPH, CHUNK), jnp.int32),
        pltpu.VMEM((CHUNK, D), _f32),
        pltpu.VMEM((CHUNK, D), _f32),
        pltpu.VMEM((_CPH, CHUNK), _f32),
        pltpu.VMEM_SHARED((NPAD, D), _f32),
        pltpu.VMEM_SHARED((NPAD,), _f32),
        pltpu.VMEM_SHARED((NPAD,), _f32),
        pltpu.SemaphoreType.DMA,
        pltpu.SemaphoreType.DMA,
        pltpu.SemaphoreType.DMA,
        pltpu.SemaphoreType.DMA,
        pltpu.SemaphoreType.DMA,
        pltpu.SemaphoreType.DMA,
    ],
    compiler_params=pltpu.CompilerParams(needs_layout_passes=False),
)(_edge_body)


def _tc2_body(acc0, acc1, xws, dp, gp, b1, w2, b2, out):
    a = acc0[...] + acc1[...] + xws[...]
    dis_row = lax.rsqrt(dp[0:1, :] + dp[1:2, :] + 1.0)
    dis_col = jnp.transpose(dis_row)
    h = jnp.maximum(dis_col * a + b1[...], 0.0)
    g_row = gp[0:1, :] + gp[1:2, :]
    s = dis_row * (g_row + dis_row) * (1.0 / N)
    cid = lax.broadcasted_iota(jnp.int32, (1, NPAD), 1)
    s = jnp.where(cid < N, s, 0.0)
    v = jnp.dot(s, h, preferred_element_type=_f32)
    out[...] = jnp.dot(v, w2[...], preferred_element_type=_f32) + b2[...]


_k4 = pl.pallas_call(
    _tc2_body,
    out_shape=jax.ShapeDtypeStruct((1, D), _f32),
)


def kernel(x, edge_index, W1, b1, W2, b2):
    row = edge_index[0]
    col = edge_index[1]
    pad = N + (jnp.arange(EPAD - E, dtype=jnp.int32) % (NPAD - N))
    row2d = jnp.concatenate([row, pad]).reshape(NW * CPW, CHUNK)
    col2d = jnp.concatenate([col, pad]).reshape(NW * CPW, CHUNK)
    zeros_n = jnp.zeros((NPAD,), _f32)
    zeros_nd = jnp.zeros((NPAD, D), _f32)

    deg_parts = _k1(col2d, zeros_n)
    dis1d, xws = _k2(x, W1, deg_parts)
    acc_parts, g_parts = _k3(row2d, col2d, xws, dis1d, zeros_nd, zeros_n)
    out2d = _k4(acc_parts[0], acc_parts[1], xws, deg_parts, g_parts,
                b1.reshape(1, D), W2, b2.reshape(1, D))
    return out2d.reshape(D)


# speedup vs baseline: 1.0614x; 1.0614x over previous
import functools

import jax
import jax.numpy as jnp
from jax import lax
from jax.experimental import pallas as pl
from jax.experimental.pallas import tpu as pltpu
from jax.experimental.pallas import tpu_sc as plsc

N = 10000
E = 320000
D = 128

NC = 2
NS = 16
NW = NC * NS
CHUNK = 64
CPW = 160
EPW = CPW * CHUNK
EPAD = NW * EPW
NPAD = 10112
RPS = NPAD // NS

_f32 = jnp.float32


def _deg_body(col2d, zeros_n, out, idx_v, ones_v, deg_s):
    cid = lax.axis_index("c")
    sid = lax.axis_index("s")
    wid = sid * NC + cid
    pltpu.sync_copy(col2d.at[pl.ds(wid * CPW, CPW)], idx_v)
    for k in range(CHUNK // 16):
        ones_v[pl.ds(k * 16, 16)] = jnp.full((16,), 1.0, _f32)

    @pl.when(sid == 0)
    def _():
        pltpu.sync_copy(zeros_n, deg_s)

    plsc.subcore_barrier()

    def body(j, c):
        pltpu.sync_copy(ones_v, deg_s.at[idx_v.at[j]], add=True)
        return c

    lax.fori_loop(0, CPW, body, 0)
    plsc.subcore_barrier()

    @pl.when(sid == 0)
    def _():
        pltpu.sync_copy(deg_s, out.at[cid])


_k1 = functools.partial(
    pl.kernel,
    out_type=jax.ShapeDtypeStruct((NC, NPAD), _f32),
    mesh=plsc.VectorSubcoreMesh(core_axis_name="c", subcore_axis_name="s"),
    scratch_types=[
        pltpu.VMEM((CPW, CHUNK), jnp.int32),
        pltpu.VMEM((CHUNK,), _f32),
        pltpu.VMEM_SHARED((NPAD,), _f32),
    ],
)(_deg_body)


def _tc1_body(x_ref, w1_ref, dp_ref, dis_ref, xws_ref):
    deg = dp_ref[0:1, :] + dp_ref[1:2, :] + 1.0
    dis_row = lax.rsqrt(deg)
    dis_ref[...] = dis_row.reshape(NPAD)
    dis_col = jnp.transpose(dis_row)
    xs = x_ref[...] * dis_col[:N]
    xw = jnp.dot(xs, w1_ref[...], preferred_element_type=_f32)
    xws_ref[0:N, :] = xw
    xws_ref[N:NPAD, :] = jnp.zeros((NPAD - N, D), _f32)


_k2 = pl.pallas_call(
    _tc1_body,
    out_shape=(
        jax.ShapeDtypeStruct((NPAD,), _f32),
        jax.ShapeDtypeStruct((NPAD, D), _f32),
    ),
)


_CPH = CPW // 4


def _edge_body(row2d, col2d, xws, dis_h, zeros_nd, zeros_n,
               acc_out, g_out,
               ridx, cidx, rows0, rows1, rows2, rows3, gvals,
               acc_s, g_s, dis_spm,
               sem0, sem1, sem2, sem3, ssem0, ssem1, ssem2, ssem3, gsem):
    cid = lax.axis_index("c")
    sid = lax.axis_index("s")
    wid = sid * NC + cid
    base = wid * CPW
    pltpu.sync_copy(zeros_nd.at[pl.ds(sid * RPS, RPS)],
                    acc_s.at[pl.ds(sid * RPS, RPS)])

    @pl.when(sid == 0)
    def _():
        pltpu.sync_copy(zeros_n, g_s)
        pltpu.sync_copy(dis_h, dis_spm)

    plsc.subcore_barrier()

    rows_b = (rows0, rows1, rows2, rows3)
    sem_b = (sem0, sem1, sem2, sem3)
    ssem_b = (ssem0, ssem1, ssem2, ssem3)
    NB = 4

    def _gather(j, b):
        pltpu.async_copy(xws.at[ridx.at[j]], rows_b[b], sem_b[b])

    def _wait_gather(j, b):
        pltpu.make_async_copy(xws.at[ridx.at[j]], rows_b[b], sem_b[b]).wait()

    def _scatter(j, b):
        pltpu.async_copy(rows_b[b], acc_s.at[cidx.at[j]], ssem_b[b],
                         add=True)

    def _wait_scatter(j, b):
        pltpu.make_async_copy(rows_b[b], acc_s.at[cidx.at[j]],
                              ssem_b[b]).wait()

    for h in range(4):
        pltpu.sync_copy(row2d.at[pl.ds(base + h * _CPH, _CPH)], ridx)
        pltpu.sync_copy(col2d.at[pl.ds(base + h * _CPH, _CPH)], cidx)
        for b in range(NB):
            _gather(b, b)

        def body(t, c):
            j0 = NB * t
            for b in range(NB):
                _wait_gather(j0 + b, b)
                _scatter(j0 + b, b)
                pltpu.async_copy(dis_spm.at[cidx.at[j0 + b]],
                                 gvals.at[j0 + b], gsem)

            @pl.when(j0 + NB < _CPH)
            def _():
                for b in range(NB):
                    _wait_scatter(j0 + b, b)
                    _gather(j0 + NB + b, b)

            return c

        lax.fori_loop(0, _CPH // NB, body, 0)
        for b in range(NB):
            _wait_scatter(_CPH - NB + b, b)

        def gdrain(j, c):
            pltpu.make_async_copy(dis_spm.at[cidx.at[j]], gvals.at[j],
                                  gsem).wait()
            pltpu.sync_copy(gvals.at[j], g_s.at[ridx.at[j]], add=True)
            return c

        lax.fori_loop(0, _CPH, gdrain, 0)

    plsc.subcore_barrier()

    pltpu.sync_copy(acc_s.at[pl.ds(sid * RPS, RPS)],
                    acc_out.at[cid, pl.ds(sid * RPS, RPS)])

    @pl.when(sid == 0)
    def _():
        pltpu.sync_copy(g_s, g_out.at[cid])


_k3 = functools.partial(
    pl.kernel,
    out_type=(
        jax.ShapeDtypeStruct((NC, NPAD, D), _f32),
        jax.ShapeDtypeStruct((NC, NPAD), _f32),
    ),
    mesh=plsc.VectorSubcoreMesh(core_axis_name="c", subcore_axis_name="s"),
    scratch_types=[
        pltpu.VMEM((_CPH, CHUNK), jnp.int32),
        pltpu.VMEM((_CPH, CHUNK), jnp.int32),
        pltpu.VMEM((CHUNK, D), _f32),
        pltpu.VMEM((CHUNK, D), _f32),
        pltpu.VMEM((CHUNK, D), _f32),
        pltpu.VMEM((CHUNK, D), _f32),
        pltpu.VMEM((_CPH, CHUNK), _f32),
        pltpu.VMEM_SHARED((NPAD, D), _f32),
        pltpu.VMEM_SHARED((NPAD,), _f32),
        pltpu.VMEM_SHARED((NPAD,), _f32),
    ] + [pltpu.SemaphoreType.DMA] * 9,
    compiler_params=pltpu.CompilerParams(needs_layout_passes=False),
)(_edge_body)


def _tc2_body(acc0, acc1, xws, dp, gp, b1, w2, b2, out):
    a = acc0[...] + acc1[...] + xws[...]
    dis_row = lax.rsqrt(dp[0:1, :] + dp[1:2, :] + 1.0)
    dis_col = jnp.transpose(dis_row)
    h = jnp.maximum(dis_col * a + b1[...], 0.0)
    g_row = gp[0:1, :] + gp[1:2, :]
    s = dis_row * (g_row + dis_row) * (1.0 / N)
    cid = lax.broadcasted_iota(jnp.int32, (1, NPAD), 1)
    s = jnp.where(cid < N, s, 0.0)
    v = jnp.dot(s, h, preferred_element_type=_f32)
    out[...] = jnp.dot(v, w2[...], preferred_element_type=_f32) + b2[...]


_k4 = pl.pallas_call(
    _tc2_body,
    out_shape=jax.ShapeDtypeStruct((1, D), _f32),
)


def kernel(x, edge_index, W1, b1, W2, b2):
    row = edge_index[0]
    col = edge_index[1]
    pad = N + (jnp.arange(EPAD - E, dtype=jnp.int32) % (NPAD - N))
    row2d = jnp.concatenate([row, pad]).reshape(NW * CPW, CHUNK)
    col2d = jnp.concatenate([col, pad]).reshape(NW * CPW, CHUNK)
    zeros_n = jnp.zeros((NPAD,), _f32)
    zeros_nd = jnp.zeros((NPAD, D), _f32)

    deg_parts = _k1(col2d, zeros_n)
    dis1d, xws = _k2(x, W1, deg_parts)
    acc_parts, g_parts = _k3(row2d, col2d, xws, dis1d, zeros_nd, zeros_n)
    out2d = _k4(acc_parts[0], acc_parts[1], xws, deg_parts, g_parts,
                b1.reshape(1, D), W2, b2.reshape(1, D))
    return out2d.reshape(D)


# speedup vs baseline: 1.3177x; 1.2414x over previous
import functools

import jax
import jax.numpy as jnp
from jax import lax
from jax.experimental import pallas as pl
from jax.experimental.pallas import tpu as pltpu
from jax.experimental.pallas import tpu_sc as plsc

N = 10000
E = 320000
D = 128

NC = 2
NS = 16
NW = NC * NS
CHUNK = 128
CPW = 80
EPW = CPW * CHUNK
EPAD = NW * EPW
NPAD = 10112
RPS = NPAD // NS

_f32 = jnp.float32


def _deg_body(col2d, zeros_n, out, idx_v, ones_v, deg_s):
    cid = lax.axis_index("c")
    sid = lax.axis_index("s")
    wid = sid * NC + cid
    pltpu.sync_copy(col2d.at[pl.ds(wid * CPW, CPW)], idx_v)
    for k in range(CHUNK // 16):
        ones_v[pl.ds(k * 16, 16)] = jnp.full((16,), 1.0, _f32)

    @pl.when(sid == 0)
    def _():
        pltpu.sync_copy(zeros_n, deg_s)

    plsc.subcore_barrier()

    def body(j, c):
        pltpu.sync_copy(ones_v, deg_s.at[idx_v.at[j]], add=True)
        return c

    lax.fori_loop(0, CPW, body, 0)
    plsc.subcore_barrier()

    @pl.when(sid == 0)
    def _():
        pltpu.sync_copy(deg_s, out.at[cid])


_k1 = functools.partial(
    pl.kernel,
    out_type=jax.ShapeDtypeStruct((NC, NPAD), _f32),
    mesh=plsc.VectorSubcoreMesh(core_axis_name="c", subcore_axis_name="s"),
    scratch_types=[
        pltpu.VMEM((CPW, CHUNK), jnp.int32),
        pltpu.VMEM((CHUNK,), _f32),
        pltpu.VMEM_SHARED((NPAD,), _f32),
    ],
)(_deg_body)


def _tc1_body(x_ref, w1_ref, dp_ref, dis_ref, xws_ref):
    deg = dp_ref[0:1, :] + dp_ref[1:2, :] + 1.0
    dis_row = lax.rsqrt(deg)
    dis_ref[...] = dis_row.reshape(NPAD)
    dis_col = jnp.transpose(dis_row)
    xs = x_ref[...] * dis_col[:N]
    xw = jnp.dot(xs, w1_ref[...], preferred_element_type=_f32)
    xws_ref[0:N, :] = xw
    xws_ref[N:NPAD, :] = jnp.zeros((NPAD - N, D), _f32)


_k2 = pl.pallas_call(
    _tc1_body,
    out_shape=(
        jax.ShapeDtypeStruct((NPAD,), _f32),
        jax.ShapeDtypeStruct((NPAD, D), _f32),
    ),
)


_CPH = CPW // 2


def _edge_body(row2d, col2d, xws, dis_h, zeros_nd, zeros_n,
               acc_out, g_out,
               ridx, cidx, rows0, rows1, gval0, gval1,
               acc_s, g_s, dis_spm,
               sem0, sem1, gsem0, gsem1):
    cid = lax.axis_index("c")
    sid = lax.axis_index("s")
    wid = sid * NC + cid
    base = wid * CPW
    pltpu.sync_copy(zeros_nd.at[pl.ds(sid * RPS, RPS)],
                    acc_s.at[pl.ds(sid * RPS, RPS)])

    @pl.when(sid == 0)
    def _():
        pltpu.sync_copy(zeros_n, g_s)
        pltpu.sync_copy(dis_h, dis_spm)

    plsc.subcore_barrier()

    rows_b = (rows0, rows1)
    sem_b = (sem0, sem1)
    gval_b = (gval0, gval1)
    gsem_b = (gsem0, gsem1)

    def _consume(j, b, nxt):
        pltpu.make_async_copy(xws.at[ridx.at[j]], rows_b[b], sem_b[b]).wait()

        @pl.when(nxt < _CPH)
        def _():
            pltpu.async_copy(xws.at[ridx.at[nxt]], rows_b[b], sem_b[b])

        pltpu.sync_copy(rows_b[b], acc_s.at[cidx.at[j]], add=True)
        pltpu.make_async_copy(dis_spm.at[cidx.at[j]], gval_b[b],
                              gsem_b[b]).wait()

        @pl.when(nxt < _CPH)
        def _():
            pltpu.async_copy(dis_spm.at[cidx.at[nxt]], gval_b[b], gsem_b[b])

        pltpu.sync_copy(gval_b[b], g_s.at[ridx.at[j]], add=True)

    for h in range(2):
        pltpu.sync_copy(row2d.at[pl.ds(base + h * _CPH, _CPH)], ridx)
        pltpu.sync_copy(col2d.at[pl.ds(base + h * _CPH, _CPH)], cidx)
        pltpu.async_copy(xws.at[ridx.at[0]], rows0, sem0)
        pltpu.async_copy(xws.at[ridx.at[1]], rows1, sem1)
        pltpu.async_copy(dis_spm.at[cidx.at[0]], gval0, gsem0)
        pltpu.async_copy(dis_spm.at[cidx.at[1]], gval1, gsem1)

        def body(t, c):
            j0 = 2 * t
            _consume(j0, 0, j0 + 2)
            _consume(j0 + 1, 1, j0 + 3)
            return c

        lax.fori_loop(0, _CPH // 2, body, 0)

    plsc.subcore_barrier()

    pltpu.sync_copy(acc_s.at[pl.ds(sid * RPS, RPS)],
                    acc_out.at[cid, pl.ds(sid * RPS, RPS)])

    @pl.when(sid == 0)
    def _():
        pltpu.sync_copy(g_s, g_out.at[cid])


_k3 = functools.partial(
    pl.kernel,
    out_type=(
        jax.ShapeDtypeStruct((NC, NPAD, D), _f32),
        jax.ShapeDtypeStruct((NC, NPAD), _f32),
    ),
    mesh=plsc.VectorSubcoreMesh(core_axis_name="c", subcore_axis_name="s"),
    scratch_types=[
        pltpu.VMEM((_CPH, CHUNK), jnp.int32),
        pltpu.VMEM((_CPH, CHUNK), jnp.int32),
        pltpu.VMEM((CHUNK, D), _f32),
        pltpu.VMEM((CHUNK, D), _f32),
        pltpu.VMEM((CHUNK,), _f32),
        pltpu.VMEM((CHUNK,), _f32),
        pltpu.VMEM_SHARED((NPAD, D), _f32),
        pltpu.VMEM_SHARED((NPAD,), _f32),
        pltpu.VMEM_SHARED((NPAD,), _f32),
    ] + [pltpu.SemaphoreType.DMA] * 4,
    compiler_params=pltpu.CompilerParams(needs_layout_passes=False),
)(_edge_body)


def _tc2_body(acc0, acc1, xws, dp, gp, b1, w2, b2, out):
    a = acc0[...] + acc1[...] + xws[...]
    dis_row = lax.rsqrt(dp[0:1, :] + dp[1:2, :] + 1.0)
    dis_col = jnp.transpose(dis_row)
    h = jnp.maximum(dis_col * a + b1[...], 0.0)
    g_row = gp[0:1, :] + gp[1:2, :]
    s = dis_row * (g_row + dis_row) * (1.0 / N)
    cid = lax.broadcasted_iota(jnp.int32, (1, NPAD), 1)
    s = jnp.where(cid < N, s, 0.0)
    v = jnp.dot(s, h, preferred_element_type=_f32)
    out[...] = jnp.dot(v, w2[...], preferred_element_type=_f32) + b2[...]


_k4 = pl.pallas_call(
    _tc2_body,
    out_shape=jax.ShapeDtypeStruct((1, D), _f32),
)


def kernel(x, edge_index, W1, b1, W2, b2):
    row = edge_index[0]
    col = edge_index[1]
    pad = N + (jnp.arange(EPAD - E, dtype=jnp.int32) % (NPAD - N))
    row2d = jnp.concatenate([row, pad]).reshape(NW * CPW, CHUNK)
    col2d = jnp.concatenate([col, pad]).reshape(NW * CPW, CHUNK)
    zeros_n = jnp.zeros((NPAD,), _f32)
    zeros_nd = jnp.zeros((NPAD, D), _f32)

    deg_parts = _k1(col2d, zeros_n)
    dis1d, xws = _k2(x, W1, deg_parts)
    acc_parts, g_parts = _k3(row2d, col2d, xws, dis1d, zeros_nd, zeros_n)
    out2d = _k4(acc_parts[0], acc_parts[1], xws, deg_parts, g_parts,
                b1.reshape(1, D), W2, b2.reshape(1, D))
    return out2d.reshape(D)


# speedup vs baseline: 1.3569x; 1.0298x over previous
import functools

import jax
import jax.numpy as jnp
from jax import lax
from jax.experimental import pallas as pl
from jax.experimental.pallas import tpu as pltpu
from jax.experimental.pallas import tpu_sc as plsc

N = 10000
E = 320000
D = 128

NC = 2
NS = 16
NW = NC * NS
CHUNK = 128
CPW = 80
EPW = CPW * CHUNK
EPAD = NW * EPW
NPAD = 10112
RPS = NPAD // NS

_f32 = jnp.float32


def _deg_body(col2d, zeros_n, out, idx_v, ones_v, deg_s, dsem):
    cid = lax.axis_index("c")
    sid = lax.axis_index("s")
    wid = sid * NC + cid
    pltpu.sync_copy(col2d.at[pl.ds(wid * CPW, CPW)], idx_v)
    for k in range(CHUNK // 16):
        ones_v[pl.ds(k * 16, 16)] = jnp.full((16,), 1.0, _f32)

    @pl.when(sid == 0)
    def _():
        pltpu.sync_copy(zeros_n, deg_s)

    plsc.subcore_barrier()

    def body(j, c):
        pltpu.async_copy(ones_v, deg_s.at[idx_v.at[j]], dsem, add=True)
        return c

    lax.fori_loop(0, CPW, body, 0)

    def drain(j, c):
        pltpu.make_async_copy(ones_v, deg_s.at[idx_v.at[j]], dsem).wait()
        return c

    lax.fori_loop(0, CPW, drain, 0)
    plsc.subcore_barrier()

    @pl.when(sid == 0)
    def _():
        pltpu.sync_copy(deg_s, out.at[cid])


_k1 = functools.partial(
    pl.kernel,
    out_type=jax.ShapeDtypeStruct((NC, NPAD), _f32),
    mesh=plsc.VectorSubcoreMesh(core_axis_name="c", subcore_axis_name="s"),
    scratch_types=[
        pltpu.VMEM((CPW, CHUNK), jnp.int32),
        pltpu.VMEM((CHUNK,), _f32),
        pltpu.VMEM_SHARED((NPAD,), _f32),
        pltpu.SemaphoreType.DMA,
    ],
)(_deg_body)


def _tc1_body(x_ref, w1_ref, dp_ref, dis_ref, xws_ref):
    deg = dp_ref[0:1, :] + dp_ref[1:2, :] + 1.0
    dis_row = lax.rsqrt(deg)
    dis_ref[...] = dis_row.reshape(NPAD)
    dis_col = jnp.transpose(dis_row)
    xs = x_ref[...] * dis_col[:N]
    xw = jnp.dot(xs, w1_ref[...], preferred_element_type=_f32)
    xws_ref[0:N, :] = xw
    xws_ref[N:NPAD, :] = jnp.zeros((NPAD - N, D), _f32)


_k2 = pl.pallas_call(
    _tc1_body,
    out_shape=(
        jax.ShapeDtypeStruct((NPAD,), _f32),
        jax.ShapeDtypeStruct((NPAD, D), _f32),
    ),
)


_CPH = CPW // 2


def _edge_body(row2d, col2d, xws, dis_h, zeros_nd, zeros_n,
               acc_out, g_out,
               ridx, cidx, rows0, rows1, gval0, gval1,
               acc_s, g_s, dis_spm,
               sem0, sem1, gsem0, gsem1):
    cid = lax.axis_index("c")
    sid = lax.axis_index("s")
    wid = sid * NC + cid
    base = wid * CPW
    pltpu.sync_copy(zeros_nd.at[pl.ds(sid * RPS, RPS)],
                    acc_s.at[pl.ds(sid * RPS, RPS)])

    @pl.when(sid == 0)
    def _():
        pltpu.sync_copy(zeros_n, g_s)
        pltpu.sync_copy(dis_h, dis_spm)

    plsc.subcore_barrier()

    rows_b = (rows0, rows1)
    sem_b = (sem0, sem1)
    gval_b = (gval0, gval1)
    gsem_b = (gsem0, gsem1)

    def _consume(j, b, nxt):
        pltpu.make_async_copy(xws.at[ridx.at[j]], rows_b[b], sem_b[b]).wait()

        @pl.when(nxt < _CPH)
        def _():
            pltpu.async_copy(xws.at[ridx.at[nxt]], rows_b[b], sem_b[b])

        pltpu.sync_copy(rows_b[b], acc_s.at[cidx.at[j]], add=True)
        pltpu.make_async_copy(dis_spm.at[cidx.at[j]], gval_b[b],
                              gsem_b[b]).wait()

        @pl.when(nxt < _CPH)
        def _():
            pltpu.async_copy(dis_spm.at[cidx.at[nxt]], gval_b[b], gsem_b[b])

        pltpu.sync_copy(gval_b[b], g_s.at[ridx.at[j]], add=True)

    for h in range(2):
        pltpu.sync_copy(row2d.at[pl.ds(base + h * _CPH, _CPH)], ridx)
        pltpu.sync_copy(col2d.at[pl.ds(base + h * _CPH, _CPH)], cidx)
        pltpu.async_copy(xws.at[ridx.at[0]], rows0, sem0)
        pltpu.async_copy(xws.at[ridx.at[1]], rows1, sem1)
        pltpu.async_copy(dis_spm.at[cidx.at[0]], gval0, gsem0)
        pltpu.async_copy(dis_spm.at[cidx.at[1]], gval1, gsem1)

        def body(t, c):
            j0 = 2 * t
            _consume(j0, 0, j0 + 2)
            _consume(j0 + 1, 1, j0 + 3)
            return c

        lax.fori_loop(0, _CPH // 2, body, 0)

    plsc.subcore_barrier()

    pltpu.sync_copy(acc_s.at[pl.ds(sid * RPS, RPS)],
                    acc_out.at[cid, pl.ds(sid * RPS, RPS)])

    @pl.when(sid == 0)
    def _():
        pltpu.sync_copy(g_s, g_out.at[cid])


_k3 = functools.partial(
    pl.kernel,
    out_type=(
        jax.ShapeDtypeStruct((NC, NPAD, D), _f32),
        jax.ShapeDtypeStruct((NC, NPAD), _f32),
    ),
    mesh=plsc.VectorSubcoreMesh(core_axis_name="c", subcore_axis_name="s"),
    scratch_types=[
        pltpu.VMEM((_CPH, CHUNK), jnp.int32),
        pltpu.VMEM((_CPH, CHUNK), jnp.int32),
        pltpu.VMEM((CHUNK, D), _f32),
        pltpu.VMEM((CHUNK, D), _f32),
        pltpu.VMEM((CHUNK,), _f32),
        pltpu.VMEM((CHUNK,), _f32),
        pltpu.VMEM_SHARED((NPAD, D), _f32),
        pltpu.VMEM_SHARED((NPAD,), _f32),
        pltpu.VMEM_SHARED((NPAD,), _f32),
    ] + [pltpu.SemaphoreType.DMA] * 4,
    compiler_params=pltpu.CompilerParams(needs_layout_passes=False),
)(_edge_body)


def _tc2_body(acc0, acc1, xws, dp, gp, b1, w2, b2, out):
    a = acc0[...] + acc1[...] + xws[...]
    dis_row = lax.rsqrt(dp[0:1, :] + dp[1:2, :] + 1.0)
    dis_col = jnp.transpose(dis_row)
    h = jnp.maximum(dis_col * a + b1[...], 0.0)
    g_row = gp[0:1, :] + gp[1:2, :]
    s = dis_row * (g_row + dis_row) * (1.0 / N)
    cid = lax.broadcasted_iota(jnp.int32, (1, NPAD), 1)
    s = jnp.where(cid < N, s, 0.0)
    v = jnp.dot(s, h, preferred_element_type=_f32)
    out[...] = jnp.dot(v, w2[...], preferred_element_type=_f32) + b2[...]


_k4 = pl.pallas_call(
    _tc2_body,
    out_shape=jax.ShapeDtypeStruct((1, D), _f32),
)


def kernel(x, edge_index, W1, b1, W2, b2):
    row = edge_index[0]
    col = edge_index[1]
    pad = N + (jnp.arange(EPAD - E, dtype=jnp.int32) % (NPAD - N))
    row2d = jnp.concatenate([row, pad]).reshape(NW * CPW, CHUNK)
    col2d = jnp.concatenate([col, pad]).reshape(NW * CPW, CHUNK)
    zeros_n = jnp.zeros((NPAD,), _f32)
    zeros_nd = jnp.zeros((NPAD, D), _f32)

    deg_parts = _k1(col2d, zeros_n)
    dis1d, xws = _k2(x, W1, deg_parts)
    acc_parts, g_parts = _k3(row2d, col2d, xws, dis1d, zeros_nd, zeros_n)
    out2d = _k4(acc_parts[0], acc_parts[1], xws, deg_parts, g_parts,
                b1.reshape(1, D), W2, b2.reshape(1, D))
    return out2d.reshape(D)


# speedup vs baseline: 1.4141x; 1.0421x over previous
import functools

import jax
import jax.numpy as jnp
from jax import lax
from jax.experimental import pallas as pl
from jax.experimental.pallas import tpu as pltpu
from jax.experimental.pallas import tpu_sc as plsc

N = 10000
E = 320000
D = 128

NC = 2
NS = 16
NW = NC * NS
CHUNK = 128
CPW = 80
EPW = CPW * CHUNK
EPAD = NW * EPW
NPAD = 10112
RPS = NPAD // NS

_f32 = jnp.float32


def _deg_body(col2d, zeros_n, out, idx_v, ones_v, deg_s, dsem):
    cid = lax.axis_index("c")
    sid = lax.axis_index("s")
    wid = sid * NC + cid
    pltpu.sync_copy(col2d.at[pl.ds(wid * CPW, CPW)], idx_v)
    for k in range(CHUNK // 16):
        ones_v[pl.ds(k * 16, 16)] = jnp.full((16,), 1.0, _f32)

    @pl.when(sid == 0)
    def _():
        pltpu.sync_copy(zeros_n, deg_s)

    plsc.subcore_barrier()

    def body(j, c):
        pltpu.async_copy(ones_v, deg_s.at[idx_v.at[j]], dsem, add=True)
        return c

    lax.fori_loop(0, CPW, body, 0)

    def drain(j, c):
        pltpu.make_async_copy(ones_v, deg_s.at[idx_v.at[j]], dsem).wait()
        return c

    lax.fori_loop(0, CPW, drain, 0)
    plsc.subcore_barrier()

    @pl.when(sid == 0)
    def _():
        pltpu.sync_copy(deg_s, out.at[cid])


_k1 = functools.partial(
    pl.kernel,
    out_type=jax.ShapeDtypeStruct((NC, NPAD), _f32),
    mesh=plsc.VectorSubcoreMesh(core_axis_name="c", subcore_axis_name="s"),
    scratch_types=[
        pltpu.VMEM((CPW, CHUNK), jnp.int32),
        pltpu.VMEM((CHUNK,), _f32),
        pltpu.VMEM_SHARED((NPAD,), _f32),
        pltpu.SemaphoreType.DMA,
    ],
)(_deg_body)


def _tc1_body(x_ref, w1_ref, dp_ref, dis_ref, xws_ref):
    deg = dp_ref[0:1, :] + dp_ref[1:2, :] + 1.0
    dis_row = lax.rsqrt(deg)
    dis_ref[...] = dis_row.reshape(NPAD)
    dis_col = jnp.transpose(dis_row)
    xs = x_ref[...] * dis_col[:N]
    xw = jnp.dot(xs, w1_ref[...], preferred_element_type=_f32)
    xws_ref[0:N, :] = xw
    xws_ref[N:NPAD, :] = jnp.zeros((NPAD - N, D), _f32)


_k2 = pl.pallas_call(
    _tc1_body,
    out_shape=(
        jax.ShapeDtypeStruct((NPAD,), _f32),
        jax.ShapeDtypeStruct((NPAD, D), _f32),
    ),
)


_CPH = CPW // 2


def _edge_body(row2d, col2d, xws, dis_h, zeros_nd, zeros_n,
               acc_out, g_out,
               ridx, cidx, rows0, rows1, gval0, gval1,
               acc_s, g_s, dis_spm,
               sem0, sem1, gsem0, gsem1):
    cid = lax.axis_index("c")
    sid = lax.axis_index("s")
    wid = sid * NC + cid
    base = wid * CPW
    pltpu.sync_copy(zeros_nd.at[pl.ds(sid * RPS, RPS)],
                    acc_s.at[pl.ds(sid * RPS, RPS)])

    @pl.when(sid == 0)
    def _():
        pltpu.sync_copy(zeros_n, g_s)
        pltpu.sync_copy(dis_h, dis_spm)

    plsc.subcore_barrier()

    rows_b = (rows0, rows1)
    sem_b = (sem0, sem1)
    gval_b = (gval0, gval1)
    gsem_b = (gsem0, gsem1)

    def _consume(j, b, nxt):
        pltpu.make_async_copy(xws.at[ridx.at[j]], rows_b[b], sem_b[b]).wait()

        @pl.when(nxt < _CPH)
        def _():
            pltpu.async_copy(xws.at[ridx.at[nxt]], rows_b[b], sem_b[b])

        pltpu.sync_copy(rows_b[b], acc_s.at[cidx.at[j]], add=True)
        pltpu.make_async_copy(dis_spm.at[cidx.at[j]], gval_b[b],
                              gsem_b[b]).wait()

        @pl.when(nxt < _CPH)
        def _():
            pltpu.async_copy(dis_spm.at[cidx.at[nxt]], gval_b[b], gsem_b[b])

        pltpu.sync_copy(gval_b[b], g_s.at[ridx.at[j]], add=True)

    for h in range(2):
        pltpu.sync_copy(row2d.at[pl.ds(base + h * _CPH, _CPH)], ridx)
        pltpu.sync_copy(col2d.at[pl.ds(base + h * _CPH, _CPH)], cidx)
        pltpu.async_copy(xws.at[ridx.at[0]], rows0, sem0)
        pltpu.async_copy(xws.at[ridx.at[1]], rows1, sem1)
        pltpu.async_copy(dis_spm.at[cidx.at[0]], gval0, gsem0)
        pltpu.async_copy(dis_spm.at[cidx.at[1]], gval1, gsem1)

        def body(t, c):
            j0 = 2 * t
            _consume(j0, 0, j0 + 2)
            _consume(j0 + 1, 1, j0 + 3)
            return c

        lax.fori_loop(0, _CPH // 2, body, 0)

    plsc.subcore_barrier()

    pltpu.sync_copy(acc_s.at[pl.ds(sid * RPS, RPS)],
                    acc_out.at[cid, pl.ds(sid * RPS, RPS)])

    @pl.when(sid == 0)
    def _():
        pltpu.sync_copy(g_s, g_out.at[cid])


_k3 = functools.partial(
    pl.kernel,
    out_type=(
        jax.ShapeDtypeStruct((NC, NPAD, D), _f32),
        jax.ShapeDtypeStruct((NC, NPAD), _f32),
    ),
    mesh=plsc.VectorSubcoreMesh(core_axis_name="c", subcore_axis_name="s"),
    scratch_types=[
        pltpu.VMEM((_CPH, CHUNK), jnp.int32),
        pltpu.VMEM((_CPH, CHUNK), jnp.int32),
        pltpu.VMEM((CHUNK, D), _f32),
        pltpu.VMEM((CHUNK, D), _f32),
        pltpu.VMEM((CHUNK,), _f32),
        pltpu.VMEM((CHUNK,), _f32),
        pltpu.VMEM_SHARED((NPAD, D), _f32),
        pltpu.VMEM_SHARED((NPAD,), _f32),
        pltpu.VMEM_SHARED((NPAD,), _f32),
    ] + [pltpu.SemaphoreType.DMA] * 4,
    compiler_params=pltpu.CompilerParams(needs_layout_passes=False),
)(_edge_body)


def _tc2_body(ap, xws, dp, gp, b1, w2, b2, out):
    a = ap[0] + ap[1] + xws[...]
    dis_row = lax.rsqrt(dp[0:1, :] + dp[1:2, :] + 1.0)
    dis_col = jnp.transpose(dis_row)
    h = jnp.maximum(dis_col * a + b1[...], 0.0)
    g_row = gp[0:1, :] + gp[1:2, :]
    s = dis_row * (g_row + dis_row) * (1.0 / N)
    cid = lax.broadcasted_iota(jnp.int32, (1, NPAD), 1)
    s = jnp.where(cid < N, s, 0.0)
    v = jnp.dot(s, h, preferred_element_type=_f32)
    out[...] = jnp.dot(v, w2[...], preferred_element_type=_f32) + b2[...]


_k4 = pl.pallas_call(
    _tc2_body,
    out_shape=jax.ShapeDtypeStruct((1, D), _f32),
)


def kernel(x, edge_index, W1, b1, W2, b2):
    row = edge_index[0]
    col = edge_index[1]
    pad = N + (jnp.arange(EPAD - E, dtype=jnp.int32) % (NPAD - N))
    row2d = jnp.concatenate([row, pad]).reshape(NW * CPW, CHUNK)
    col2d = jnp.concatenate([col, pad]).reshape(NW * CPW, CHUNK)
    zeros_n = jnp.zeros((NPAD,), _f32)
    zeros_nd = jnp.zeros((NPAD, D), _f32)

    deg_parts = _k1(col2d, zeros_n)
    dis1d, xws = _k2(x, W1, deg_parts)
    acc_parts, g_parts = _k3(row2d, col2d, xws, dis1d, zeros_nd, zeros_n)
    out2d = _k4(acc_parts, xws, deg_parts, g_parts,
                b1.reshape(1, D), W2, b2.reshape(1, D))
    return out2d.reshape(D)


# speedup vs baseline: 1.4257x; 1.0082x over previous
import functools

import jax
import jax.numpy as jnp
from jax import lax
from jax.experimental import pallas as pl
from jax.experimental.pallas import tpu as pltpu
from jax.experimental.pallas import tpu_sc as plsc

N = 10000
E = 320000
D = 128

NC = 2
NS = 16
NW = NC * NS
CHUNK = 128
CPW = 80
EPW = CPW * CHUNK
EPAD = NW * EPW
NPAD = 10112
RPS = NPAD // NS

_f32 = jnp.float32


def _deg_body(col1d, zeros_n, out, idx_v, ones_v, deg_s, dsem, stsem):
    cid = lax.axis_index("c")
    sid = lax.axis_index("s")
    wid = sid * NC + cid
    base = wid * CPW

    def stage(j, c):
        pltpu.async_copy(col1d.at[pl.ds((base + j) * CHUNK, CHUNK)],
                         idx_v.at[j], stsem)
        return c

    lax.fori_loop(0, CPW, stage, 0)

    def stage_drain(j, c):
        pltpu.make_async_copy(col1d.at[pl.ds((base + j) * CHUNK, CHUNK)],
                              idx_v.at[j], stsem).wait()
        return c

    lax.fori_loop(0, CPW, stage_drain, 0)
    for k in range(CHUNK // 16):
        ones_v[pl.ds(k * 16, 16)] = jnp.full((16,), 1.0, _f32)

    @pl.when(sid == 0)
    def _():
        pltpu.sync_copy(zeros_n, deg_s)

    plsc.subcore_barrier()

    def body(j, c):
        pltpu.async_copy(ones_v, deg_s.at[idx_v.at[j]], dsem, add=True)
        return c

    lax.fori_loop(0, CPW, body, 0)

    def drain(j, c):
        pltpu.make_async_copy(ones_v, deg_s.at[idx_v.at[j]], dsem).wait()
        return c

    lax.fori_loop(0, CPW, drain, 0)
    plsc.subcore_barrier()

    @pl.when(sid == 0)
    def _():
        pltpu.sync_copy(deg_s, out.at[cid])


_k1 = functools.partial(
    pl.kernel,
    out_type=jax.ShapeDtypeStruct((NC, NPAD), _f32),
    mesh=plsc.VectorSubcoreMesh(core_axis_name="c", subcore_axis_name="s"),
    scratch_types=[
        pltpu.VMEM((CPW, CHUNK), jnp.int32),
        pltpu.VMEM((CHUNK,), _f32),
        pltpu.VMEM_SHARED((NPAD,), _f32),
        pltpu.SemaphoreType.DMA,
        pltpu.SemaphoreType.DMA,
    ],
)(_deg_body)


def _tc1_body(x_ref, w1_ref, dp_ref, dis_ref, xws_ref):
    deg = dp_ref[0:1, :] + dp_ref[1:2, :] + 1.0
    dis_row = lax.rsqrt(deg)
    dis_ref[...] = dis_row.reshape(NPAD)
    dis_col = jnp.transpose(dis_row)
    xs = x_ref[...] * dis_col[:N]
    xw = jnp.dot(xs, w1_ref[...], preferred_element_type=_f32)
    xws_ref[0:N, :] = xw
    xws_ref[N:NPAD, :] = jnp.zeros((NPAD - N, D), _f32)


_k2 = pl.pallas_call(
    _tc1_body,
    out_shape=(
        jax.ShapeDtypeStruct((NPAD,), _f32),
        jax.ShapeDtypeStruct((NPAD, D), _f32),
    ),
)


_CPH = CPW // 2


def _edge_body(row1d, col1d, xws, dis_h, zeros_nd, zeros_n,
               acc_out, g_out,
               ridx, cidx, rows0, rows1, gval0, gval1,
               acc_s, g_s, dis_spm,
               sem0, sem1, gsem0, gsem1, stsem, stsem2):
    cid = lax.axis_index("c")
    sid = lax.axis_index("s")
    wid = sid * NC + cid
    base = wid * CPW
    pltpu.sync_copy(zeros_nd.at[pl.ds(sid * RPS, RPS)],
                    acc_s.at[pl.ds(sid * RPS, RPS)])

    @pl.when(sid == 0)
    def _():
        pltpu.sync_copy(zeros_n, g_s)
        pltpu.sync_copy(dis_h, dis_spm)

    plsc.subcore_barrier()

    rows_b = (rows0, rows1)
    sem_b = (sem0, sem1)
    gval_b = (gval0, gval1)
    gsem_b = (gsem0, gsem1)

    def _consume(j, b, nxt):
        pltpu.make_async_copy(xws.at[ridx.at[j]], rows_b[b], sem_b[b]).wait()

        @pl.when(nxt < _CPH)
        def _():
            pltpu.async_copy(xws.at[ridx.at[nxt]], rows_b[b], sem_b[b])

        pltpu.sync_copy(rows_b[b], acc_s.at[cidx.at[j]], add=True)
        pltpu.make_async_copy(dis_spm.at[cidx.at[j]], gval_b[b],
                              gsem_b[b]).wait()

        @pl.when(nxt < _CPH)
        def _():
            pltpu.async_copy(dis_spm.at[cidx.at[nxt]], gval_b[b], gsem_b[b])

        pltpu.sync_copy(gval_b[b], g_s.at[ridx.at[j]], add=True)

    for h in range(2):
        hbase = (base + h * _CPH) * CHUNK

        def stage(j, c):
            pltpu.async_copy(row1d.at[pl.ds(hbase + j * CHUNK, CHUNK)],
                             ridx.at[j], stsem)
            pltpu.async_copy(col1d.at[pl.ds(hbase + j * CHUNK, CHUNK)],
                             cidx.at[j], stsem2)
            return c

        lax.fori_loop(0, _CPH, stage, 0)

        def stage_drain(j, c):
            pltpu.make_async_copy(row1d.at[pl.ds(hbase + j * CHUNK, CHUNK)],
                                  ridx.at[j], stsem).wait()
            pltpu.make_async_copy(col1d.at[pl.ds(hbase + j * CHUNK, CHUNK)],
                                  cidx.at[j], stsem2).wait()
            return c

        lax.fori_loop(0, _CPH, stage_drain, 0)
        pltpu.async_copy(xws.at[ridx.at[0]], rows0, sem0)
        pltpu.async_copy(xws.at[ridx.at[1]], rows1, sem1)
        pltpu.async_copy(dis_spm.at[cidx.at[0]], gval0, gsem0)
        pltpu.async_copy(dis_spm.at[cidx.at[1]], gval1, gsem1)

        def body(t, c):
            j0 = 2 * t
            _consume(j0, 0, j0 + 2)
            _consume(j0 + 1, 1, j0 + 3)
            return c

        lax.fori_loop(0, _CPH // 2, body, 0)

    plsc.subcore_barrier()

    pltpu.sync_copy(acc_s.at[pl.ds(sid * RPS, RPS)],
                    acc_out.at[cid, pl.ds(sid * RPS, RPS)])

    @pl.when(sid == 0)
    def _():
        pltpu.sync_copy(g_s, g_out.at[cid])


_k3 = functools.partial(
    pl.kernel,
    out_type=(
        jax.ShapeDtypeStruct((NC, NPAD, D), _f32),
        jax.ShapeDtypeStruct((NC, NPAD), _f32),
    ),
    mesh=plsc.VectorSubcoreMesh(core_axis_name="c", subcore_axis_name="s"),
    scratch_types=[
        pltpu.VMEM((_CPH, CHUNK), jnp.int32),
        pltpu.VMEM((_CPH, CHUNK), jnp.int32),
        pltpu.VMEM((CHUNK, D), _f32),
        pltpu.VMEM((CHUNK, D), _f32),
        pltpu.VMEM((CHUNK,), _f32),
        pltpu.VMEM((CHUNK,), _f32),
        pltpu.VMEM_SHARED((NPAD, D), _f32),
        pltpu.VMEM_SHARED((NPAD,), _f32),
        pltpu.VMEM_SHARED((NPAD,), _f32),
    ] + [pltpu.SemaphoreType.DMA] * 6,
    compiler_params=pltpu.CompilerParams(needs_layout_passes=False),
)(_edge_body)


def _tc2_body(ap, xws, dp, gp, b1, w2, b2, out):
    a = ap[0] + ap[1] + xws[...]
    dis_row = lax.rsqrt(dp[0:1, :] + dp[1:2, :] + 1.0)
    dis_col = jnp.transpose(dis_row)
    h = jnp.maximum(dis_col * a + b1[...], 0.0)
    g_row = gp[0:1, :] + gp[1:2, :]
    s = dis_row * (g_row + dis_row) * (1.0 / N)
    cid = lax.broadcasted_iota(jnp.int32, (1, NPAD), 1)
    s = jnp.where(cid < N, s, 0.0)
    v = jnp.dot(s, h, preferred_element_type=_f32)
    out[...] = jnp.dot(v, w2[...], preferred_element_type=_f32) + b2[...]


_k4 = pl.pallas_call(
    _tc2_body,
    out_shape=jax.ShapeDtypeStruct((1, D), _f32),
)


def kernel(x, edge_index, W1, b1, W2, b2):
    row = edge_index[0]
    col = edge_index[1]
    pad = N + (jnp.arange(EPAD - E, dtype=jnp.int32) % (NPAD - N))
    row1d = jnp.concatenate([row, pad])
    col1d = jnp.concatenate([col, pad])
    zeros_n = jnp.zeros((NPAD,), _f32)
    zeros_nd = jnp.zeros((NPAD, D), _f32)

    deg_parts = _k1(col1d, zeros_n)
    dis1d, xws = _k2(x, W1, deg_parts)
    acc_parts, g_parts = _k3(row1d, col1d, xws, dis1d, zeros_nd, zeros_n)
    out2d = _k4(acc_parts, xws, deg_parts, g_parts,
                b1.reshape(1, D), W2, b2.reshape(1, D))
    return out2d.reshape(D)


# speedup vs baseline: 1.5548x; 1.0905x over previous
import functools

import jax
import jax.numpy as jnp
from jax import lax
from jax.experimental import pallas as pl
from jax.experimental.pallas import tpu as pltpu
from jax.experimental.pallas import tpu_sc as plsc

N = 10000
E = 320000
D = 128

NC = 2
NS = 16
NW = NC * NS
CHUNK = 128
CPW = 80
EPW = CPW * CHUNK
EPAD = NW * EPW
NPAD = 10112
RPS = NPAD // NS
ECH = E // CHUNK

_f32 = jnp.float32


def _deg_body(ei, pad1d, zeros_n, out, idx_v, ones_v, deg_s, dsem, stsem):
    cid = lax.axis_index("c")
    sid = lax.axis_index("s")
    wid = sid * NC + cid
    base = wid * CPW

    def stage(j, c):
        gc = base + j

        @pl.when(gc < ECH)
        def _():
            pltpu.async_copy(ei.at[1, pl.ds(gc * CHUNK, CHUNK)],
                             idx_v.at[j], stsem)

        @pl.when(gc >= ECH)
        def _():
            pltpu.async_copy(pad1d.at[pl.ds((gc - ECH) * CHUNK, CHUNK)],
                             idx_v.at[j], stsem)

        return c

    lax.fori_loop(0, CPW, stage, 0)

    def stage_drain(j, c):
        gc = base + j

        @pl.when(gc < ECH)
        def _():
            pltpu.make_async_copy(ei.at[1, pl.ds(gc * CHUNK, CHUNK)],
                                  idx_v.at[j], stsem).wait()

        @pl.when(gc >= ECH)
        def _():
            pltpu.make_async_copy(
                pad1d.at[pl.ds((gc - ECH) * CHUNK, CHUNK)],
                idx_v.at[j], stsem).wait()

        return c

    lax.fori_loop(0, CPW, stage_drain, 0)
    for k in range(CHUNK // 16):
        ones_v[pl.ds(k * 16, 16)] = jnp.full((16,), 1.0, _f32)

    @pl.when(sid == 0)
    def _():
        pltpu.sync_copy(zeros_n, deg_s)

    plsc.subcore_barrier()

    def body(j, c):
        pltpu.async_copy(ones_v, deg_s.at[idx_v.at[j]], dsem, add=True)
        return c

    lax.fori_loop(0, CPW, body, 0)

    def drain(j, c):
        pltpu.make_async_copy(ones_v, deg_s.at[idx_v.at[j]], dsem).wait()
        return c

    lax.fori_loop(0, CPW, drain, 0)
    plsc.subcore_barrier()

    @pl.when(sid == 0)
    def _():
        pltpu.sync_copy(deg_s, out.at[cid])


_k1 = functools.partial(
    pl.kernel,
    out_type=jax.ShapeDtypeStruct((NC, NPAD), _f32),
    mesh=plsc.VectorSubcoreMesh(core_axis_name="c", subcore_axis_name="s"),
    scratch_types=[
        pltpu.VMEM((CPW, CHUNK), jnp.int32),
        pltpu.VMEM((CHUNK,), _f32),
        pltpu.VMEM_SHARED((NPAD,), _f32),
        pltpu.SemaphoreType.DMA,
        pltpu.SemaphoreType.DMA,
    ],
)(_deg_body)


def _tc1_body(x_ref, w1_ref, dp_ref, dis_ref, xws_ref):
    deg = dp_ref[0:1, :] + dp_ref[1:2, :] + 1.0
    dis_row = lax.rsqrt(deg)
    dis_ref[...] = dis_row.reshape(NPAD)
    dis_col = jnp.transpose(dis_row)
    xs = x_ref[...] * dis_col[:N]
    xw = jnp.dot(xs, w1_ref[...], preferred_element_type=_f32)
    xws_ref[0:N, :] = xw
    xws_ref[N:NPAD, :] = jnp.zeros((NPAD - N, D), _f32)


_k2 = pl.pallas_call(
    _tc1_body,
    out_shape=(
        jax.ShapeDtypeStruct((NPAD,), _f32),
        jax.ShapeDtypeStruct((NPAD, D), _f32),
    ),
)


_CPH = CPW // 2


def _edge_body(ei, pad1d, xws, dis_h, zeros_nd, zeros_n,
               acc_out, g_out,
               ridx, cidx, rows0, rows1, gval0, gval1,
               acc_s, g_s, dis_spm,
               sem0, sem1, gsem0, gsem1, stsem, stsem2):
    cid = lax.axis_index("c")
    sid = lax.axis_index("s")
    wid = sid * NC + cid
    base = wid * CPW
    pltpu.sync_copy(zeros_nd.at[pl.ds(sid * RPS, RPS)],
                    acc_s.at[pl.ds(sid * RPS, RPS)])

    @pl.when(sid == 0)
    def _():
        pltpu.sync_copy(zeros_n, g_s)
        pltpu.sync_copy(dis_h, dis_spm)

    plsc.subcore_barrier()

    rows_b = (rows0, rows1)
    sem_b = (sem0, sem1)
    gval_b = (gval0, gval1)
    gsem_b = (gsem0, gsem1)

    def _consume(j, b, nxt):
        pltpu.make_async_copy(xws.at[ridx.at[j]], rows_b[b], sem_b[b]).wait()

        @pl.when(nxt < _CPH)
        def _():
            pltpu.async_copy(xws.at[ridx.at[nxt]], rows_b[b], sem_b[b])

        pltpu.sync_copy(rows_b[b], acc_s.at[cidx.at[j]], add=True)
        pltpu.make_async_copy(dis_spm.at[cidx.at[j]], gval_b[b],
                              gsem_b[b]).wait()

        @pl.when(nxt < _CPH)
        def _():
            pltpu.async_copy(dis_spm.at[cidx.at[nxt]], gval_b[b], gsem_b[b])

        pltpu.sync_copy(gval_b[b], g_s.at[ridx.at[j]], add=True)

    for h in range(2):
        hb = base + h * _CPH

        def stage(j, c):
            gc = hb + j

            @pl.when(gc < ECH)
            def _():
                pltpu.async_copy(ei.at[0, pl.ds(gc * CHUNK, CHUNK)],
                                 ridx.at[j], stsem)
                pltpu.async_copy(ei.at[1, pl.ds(gc * CHUNK, CHUNK)],
                                 cidx.at[j], stsem2)

            @pl.when(gc >= ECH)
            def _():
                po = (gc - ECH) * CHUNK
                pltpu.async_copy(pad1d.at[pl.ds(po, CHUNK)], ridx.at[j],
                                 stsem)
                pltpu.async_copy(pad1d.at[pl.ds(po, CHUNK)], cidx.at[j],
                                 stsem2)

            return c

        lax.fori_loop(0, _CPH, stage, 0)

        def stage_drain(j, c):
            gc = hb + j

            @pl.when(gc < ECH)
            def _():
                pltpu.make_async_copy(ei.at[0, pl.ds(gc * CHUNK, CHUNK)],
                                      ridx.at[j], stsem).wait()
                pltpu.make_async_copy(ei.at[1, pl.ds(gc * CHUNK, CHUNK)],
                                      cidx.at[j], stsem2).wait()

            @pl.when(gc >= ECH)
            def _():
                po = (gc - ECH) * CHUNK
                pltpu.make_async_copy(pad1d.at[pl.ds(po, CHUNK)],
                                      ridx.at[j], stsem).wait()
                pltpu.make_async_copy(pad1d.at[pl.ds(po, CHUNK)],
                                      cidx.at[j], stsem2).wait()

            return c

        lax.fori_loop(0, _CPH, stage_drain, 0)
        pltpu.async_copy(xws.at[ridx.at[0]], rows0, sem0)
        pltpu.async_copy(xws.at[ridx.at[1]], rows1, sem1)
        pltpu.async_copy(dis_spm.at[cidx.at[0]], gval0, gsem0)
        pltpu.async_copy(dis_spm.at[cidx.at[1]], gval1, gsem1)

        def body(t, c):
            j0 = 2 * t
            _consume(j0, 0, j0 + 2)
            _consume(j0 + 1, 1, j0 + 3)
            return c

        lax.fori_loop(0, _CPH // 2, body, 0)

    plsc.subcore_barrier()

    pltpu.sync_copy(acc_s.at[pl.ds(sid * RPS, RPS)],
                    acc_out.at[cid, pl.ds(sid * RPS, RPS)])

    @pl.when(sid == 0)
    def _():
        pltpu.sync_copy(g_s, g_out.at[cid])


_k3 = functools.partial(
    pl.kernel,
    out_type=(
        jax.ShapeDtypeStruct((NC, NPAD, D), _f32),
        jax.ShapeDtypeStruct((NC, NPAD), _f32),
    ),
    mesh=plsc.VectorSubcoreMesh(core_axis_name="c", subcore_axis_name="s"),
    scratch_types=[
        pltpu.VMEM((_CPH, CHUNK), jnp.int32),
        pltpu.VMEM((_CPH, CHUNK), jnp.int32),
        pltpu.VMEM((CHUNK, D), _f32),
        pltpu.VMEM((CHUNK, D), _f32),
        pltpu.VMEM((CHUNK,), _f32),
        pltpu.VMEM((CHUNK,), _f32),
        pltpu.VMEM_SHARED((NPAD, D), _f32),
        pltpu.VMEM_SHARED((NPAD,), _f32),
        pltpu.VMEM_SHARED((NPAD,), _f32),
    ] + [pltpu.SemaphoreType.DMA] * 6,
    compiler_params=pltpu.CompilerParams(needs_layout_passes=False),
)(_edge_body)


def _tc2_body(ap, xws, dp, gp, b1, w2, b2, out):
    a = ap[0] + ap[1] + xws[...]
    dis_row = lax.rsqrt(dp[0:1, :] + dp[1:2, :] + 1.0)
    dis_col = jnp.transpose(dis_row)
    h = jnp.maximum(dis_col * a + b1[...], 0.0)
    g_row = gp[0:1, :] + gp[1:2, :]
    s = dis_row * (g_row + dis_row) * (1.0 / N)
    cid = lax.broadcasted_iota(jnp.int32, (1, NPAD), 1)
    s = jnp.where(cid < N, s, 0.0)
    v = jnp.dot(s, h, preferred_element_type=_f32)
    out[...] = jnp.dot(v, w2[...], preferred_element_type=_f32) + b2[...]


_k4 = pl.pallas_call(
    _tc2_body,
    out_shape=jax.ShapeDtypeStruct((1, D), _f32),
)


def kernel(x, edge_index, W1, b1, W2, b2):

    pad1d = N + (jnp.arange(EPAD - E, dtype=jnp.int32) % (NPAD - N))
    zeros_n = jnp.zeros((NPAD,), _f32)
    zeros_nd = jnp.zeros((NPAD, D), _f32)

    deg_parts = _k1(edge_index, pad1d, zeros_n)
    dis1d, xws = _k2(x, W1, deg_parts)
    acc_parts, g_parts = _k3(edge_index, pad1d, xws, dis1d, zeros_nd, zeros_n)
    out2d = _k4(acc_parts, xws, deg_parts, g_parts,
                b1.reshape(1, D), W2, b2.reshape(1, D))
    return out2d.reshape(D)
